# baseline (speedup 1.0000x reference)
import jax
import jax.numpy as jnp
from jax import lax
from jax.experimental import pallas as pl
from jax.experimental.pallas import tpu as pltpu

N_HEADS = 4
SIGMA = 6.0
ALPHA = 0.2


def _gcn_kernel(x_ref, w_ref, b_ref, d_ref, nbr_ref, out_ref, a_scr, prev_scr):
    b_id = pl.program_id(0)
    t_id = pl.program_id(1)

    @pl.when(jnp.logical_and(b_id == 0, t_id == 0))
    def _build_a():
        d = d_ref[...]
        nbr = nbr_ref[...]
        n = d.shape[0]
        n_iota = lax.broadcasted_iota(jnp.int32, (n, n), 1)
        for h in range(N_HEADS):
            lam_h = (h + 1) / N_HEADS
            acc = jnp.zeros((n, n), dtype=jnp.float32)
            for k in range(d.shape[1]):
                w_col = jnp.exp(-(d[:, k:k + 1] ** 2) * (lam_h / (SIGMA * SIGMA)))
                acc = acc + w_col * (nbr[:, k:k + 1] == n_iota).astype(jnp.float32)
            a_scr[h] = acc

    x = x_ref[0, 0]
    y = lax.dot_general(x, w_ref[...], (((1,), (1,)), ((), ())),
                        preferred_element_type=jnp.float32)

    bias = b_ref[0]
    for h in range(N_HEADS):
        agg = jnp.dot(a_scr[h], y, preferred_element_type=jnp.float32)
        prev = prev_scr[h]
        sm = jnp.where(t_id == 0, agg, (1.0 - ALPHA) * agg + ALPHA * prev)
        out_ref[0, 0, :, h, :] = jnp.maximum(sm + bias[None, :], 0.0)
        prev_scr[h] = agg


def kernel(x, W, b, dists, neighbors):
    B, T, N, D = x.shape
    H = N_HEADS
    b2 = b.reshape(1, D)
    grid = (B, T)
    out = pl.pallas_call(
        _gcn_kernel,
        grid=grid,
        in_specs=[
            pl.BlockSpec((1, 1, N, D), lambda b_, t_: (b_, t_, 0, 0)),
            pl.BlockSpec((D, D), lambda b_, t_: (0, 0)),
            pl.BlockSpec((1, D), lambda b_, t_: (0, 0)),
            pl.BlockSpec(dists.shape, lambda b_, t_: (0, 0)),
            pl.BlockSpec(neighbors.shape, lambda b_, t_: (0, 0)),
        ],
        out_specs=pl.BlockSpec((1, 1, N, H, D), lambda b_, t_: (b_, t_, 0, 0, 0)),
        out_shape=jax.ShapeDtypeStruct((B, T, N, H, D), jnp.float32),
        scratch_shapes=[
            pltpu.VMEM((H, N, N), jnp.float32),
            pltpu.VMEM((H, N, D), jnp.float32),
        ],
    )(x, W, b2, dists, neighbors)
    return out


# speedup vs baseline: 3.3718x; 3.3718x over previous
import jax
import jax.numpy as jnp
from jax import lax
from jax.experimental import pallas as pl
from jax.experimental.pallas import tpu as pltpu

N_HEADS = 4
SIGMA = 6.0
ALPHA = 0.2
TCHUNK = 8


def _gcn_kernel(T, x_ref, w_ref, b_ref, d_ref, nbr_ref, out_ref, a_scr, prev_scr):
    c = pl.program_id(0)
    N, K = d_ref.shape
    NH = N * N_HEADS

    @pl.when(c == 0)
    def _build_a():
        r_row = lax.broadcasted_iota(jnp.int32, (NH, N), 0)
        i_col = lax.broadcasted_iota(jnp.int32, (NH, N), 1)
        rep = ((r_row // N_HEADS) == i_col).astype(jnp.float32)
        d_rep = jnp.dot(rep, d_ref[...], preferred_element_type=jnp.float32)
        nbr_rep = jnp.dot(rep, nbr_ref[...].astype(jnp.float32),
                          preferred_element_type=jnp.float32)
        lam = ((lax.broadcasted_iota(jnp.int32, (NH, 1), 0) % N_HEADS) + 1
               ).astype(jnp.float32) * (1.0 / N_HEADS)
        n_f = lax.broadcasted_iota(jnp.int32, (NH, N), 1).astype(jnp.float32)
        acc = jnp.zeros((NH, N), dtype=jnp.float32)
        inv_s2 = 1.0 / (SIGMA * SIGMA)
        for k in range(K):
            wgt = jnp.exp(-(d_rep[:, k:k + 1] ** 2) * lam * inv_s2)
            acc = acc + wgt * (nbr_rep[:, k:k + 1] == n_f).astype(jnp.float32)
        a_scr[...] = acc

    a_big = a_scr[...]
    w_mat = w_ref[...]
    bias = b_ref[0]
    prev = prev_scr[...]
    for t in range(TCHUNK):
        x_t = x_ref[t]
        y = lax.dot_general(x_t, w_mat, (((1,), (1,)), ((), ())),
                            preferred_element_type=jnp.float32)
        agg = jnp.dot(a_big, y, preferred_element_type=jnp.float32)
        is_start = (c * TCHUNK + t) % T == 0
        sm = jnp.where(is_start, agg, (1.0 - ALPHA) * agg + ALPHA * prev)
        out_ref[t] = jnp.maximum(sm + bias[None, :], 0.0)
        prev = agg
    prev_scr[...] = prev


def kernel(x, W, b, dists, neighbors):
    B, T, N, D = x.shape
    H = N_HEADS
    NH = N * H
    xr = x.reshape(B * T, N, D)
    b2 = b.reshape(1, D)
    n_chunks = (B * T) // TCHUNK

    import functools
    body = functools.partial(_gcn_kernel, T)
    out = pl.pallas_call(
        body,
        grid=(n_chunks,),
        in_specs=[
            pl.BlockSpec((TCHUNK, N, D), lambda c: (c, 0, 0)),
            pl.BlockSpec((D, D), lambda c: (0, 0)),
            pl.BlockSpec((1, D), lambda c: (0, 0)),
            pl.BlockSpec(dists.shape, lambda c: (0, 0)),
            pl.BlockSpec(neighbors.shape, lambda c: (0, 0)),
        ],
        out_specs=pl.BlockSpec((TCHUNK, NH, D), lambda c: (c, 0, 0)),
        out_shape=jax.ShapeDtypeStruct((B * T, NH, D), jnp.float32),
        scratch_shapes=[
            pltpu.VMEM((NH, N), jnp.float32),
            pltpu.VMEM((NH, D), jnp.float32),
        ],
    )(xr, W, b2, dists, neighbors)
    return out.reshape(B, T, N, H, D)


# speedup vs baseline: 5.1363x; 1.5233x over previous
import functools

import jax
import jax.numpy as jnp
from jax import lax
from jax.experimental import pallas as pl
from jax.experimental.pallas import tpu as pltpu

N_HEADS = 4
SIGMA = 6.0
ALPHA = 0.2
TCHUNK = 8


def _gcn_kernel(T, x_ref, w_ref, b_ref, d_ref, nbr_ref, out_ref,
                a_scr, prev_scr, y_scr, agg_scr):
    c = pl.program_id(0)
    N, K = d_ref.shape
    NH = N * N_HEADS
    D = w_ref.shape[0]

    @pl.when(c == 0)
    def _build_a():
        r_row = lax.broadcasted_iota(jnp.int32, (NH, N), 0)
        i_col = lax.broadcasted_iota(jnp.int32, (NH, N), 1)
        rep = ((r_row // N_HEADS) == i_col).astype(jnp.float32)
        d_rep = jnp.dot(rep, d_ref[...], preferred_element_type=jnp.float32)
        nbr_rep = jnp.dot(rep, nbr_ref[...].astype(jnp.float32),
                          preferred_element_type=jnp.float32)
        lam = ((lax.broadcasted_iota(jnp.int32, (NH, 1), 0) % N_HEADS) + 1
               ).astype(jnp.float32) * (1.0 / N_HEADS)
        n_f = lax.broadcasted_iota(jnp.int32, (NH, N), 1).astype(jnp.float32)
        acc = jnp.zeros((NH, N), dtype=jnp.float32)
        inv_s2 = 1.0 / (SIGMA * SIGMA)
        for k in range(K):
            wgt = jnp.exp(-(d_rep[:, k:k + 1] ** 2) * lam * inv_s2)
            acc = acc + wgt * (nbr_rep[:, k:k + 1] == n_f).astype(jnp.float32)
        a_scr[...] = acc

    x_all = x_ref[...].reshape(TCHUNK * N, D)
    y_stack = lax.dot_general(x_all, w_ref[...], (((1,), (1,)), ((), ())),
                              preferred_element_type=jnp.float32)
    for t in range(TCHUNK):
        y_scr[:, t * D:(t + 1) * D] = y_stack[t * N:(t + 1) * N, :]

    agg_scr[...] = jnp.dot(a_scr[...], y_scr[...],
                           preferred_element_type=jnp.float32)

    bias = b_ref[0]
    is_start = (c * TCHUNK) % T == 0
    prev = prev_scr[...]
    for t in range(TCHUNK):
        cur = agg_scr[:, t * D:(t + 1) * D]
        sm = (1.0 - ALPHA) * cur + ALPHA * prev
        if t == 0:
            sm = jnp.where(is_start, cur, sm)
        out_ref[t] = jnp.maximum(sm + bias[None, :], 0.0)
        prev = cur
    prev_scr[...] = prev


def kernel(x, W, b, dists, neighbors):
    B, T, N, D = x.shape
    H = N_HEADS
    NH = N * H
    xr = x.reshape(B * T, N, D)
    b2 = b.reshape(1, D)
    n_chunks = (B * T) // TCHUNK

    body = functools.partial(_gcn_kernel, T)
    out = pl.pallas_call(
        body,
        grid=(n_chunks,),
        in_specs=[
            pl.BlockSpec((TCHUNK, N, D), lambda c: (c, 0, 0)),
            pl.BlockSpec((D, D), lambda c: (0, 0)),
            pl.BlockSpec((1, D), lambda c: (0, 0)),
            pl.BlockSpec(dists.shape, lambda c: (0, 0)),
            pl.BlockSpec(neighbors.shape, lambda c: (0, 0)),
        ],
        out_specs=pl.BlockSpec((TCHUNK, NH, D), lambda c: (c, 0, 0)),
        out_shape=jax.ShapeDtypeStruct((B * T, NH, D), jnp.float32),
        scratch_shapes=[
            pltpu.VMEM((NH, N), jnp.float32),
            pltpu.VMEM((NH, D), jnp.float32),
            pltpu.VMEM((N, TCHUNK * D), jnp.float32),
            pltpu.VMEM((NH, TCHUNK * D), jnp.float32),
        ],
    )(xr, W, b2, dists, neighbors)
    return out.reshape(B, T, N, H, D)


# speedup vs baseline: 7.3178x; 1.4247x over previous
import functools

import jax
import jax.numpy as jnp
from jax import lax
from jax.experimental import pallas as pl
from jax.experimental.pallas import tpu as pltpu

N_HEADS = 4
SIGMA = 6.0
ALPHA = 0.2
TCHUNK = 16


def _gcn_kernel(T, x_ref, w_ref, b_ref, d_ref, nbr_ref, out_ref,
                a_scr, prev_scr, y_scr, agg_scr):
    c = pl.program_id(0)
    N, K = d_ref.shape
    NH = N * N_HEADS
    D = w_ref.shape[0]

    @pl.when(c == 0)
    def _build_a():
        r_row = lax.broadcasted_iota(jnp.int32, (NH, N), 0)
        i_col = lax.broadcasted_iota(jnp.int32, (NH, N), 1)
        rep = ((r_row // N_HEADS) == i_col).astype(jnp.float32)
        d_rep = jnp.dot(rep, d_ref[...], preferred_element_type=jnp.float32)
        nbr_rep = jnp.dot(rep, nbr_ref[...].astype(jnp.float32),
                          preferred_element_type=jnp.float32)
        lam = ((lax.broadcasted_iota(jnp.int32, (NH, 1), 0) % N_HEADS) + 1
               ).astype(jnp.float32) * (1.0 / N_HEADS)
        n_f = lax.broadcasted_iota(jnp.int32, (NH, N), 1).astype(jnp.float32)
        acc = jnp.zeros((NH, N), dtype=jnp.float32)
        inv_s2 = 1.0 / (SIGMA * SIGMA)
        for k in range(K):
            wgt = jnp.exp(-(d_rep[:, k:k + 1] ** 2) * lam * inv_s2)
            acc = acc + wgt * (nbr_rep[:, k:k + 1] == n_f).astype(jnp.float32)
        a_scr[...] = acc

    x_all = x_ref[...].reshape(TCHUNK * N, D)
    y_stack = lax.dot_general(x_all, w_ref[...], (((1,), (1,)), ((), ())),
                              preferred_element_type=jnp.float32)
    for t in range(TCHUNK):
        y_scr[:, t * D:(t + 1) * D] = y_stack[t * N:(t + 1) * N, :]

    agg_scr[...] = jnp.dot(a_scr[...], y_scr[...],
                           preferred_element_type=jnp.float32)

    bias = b_ref[0]
    is_start = (c * TCHUNK) % T == 0
    prev = prev_scr[...]
    for t in range(TCHUNK):
        cur = agg_scr[:, t * D:(t + 1) * D]
        sm = (1.0 - ALPHA) * cur + ALPHA * prev
        if t == 0:
            sm = jnp.where(is_start, cur, sm)
        out_ref[t] = jnp.maximum(sm + bias[None, :], 0.0)
        prev = cur
    prev_scr[...] = prev


def kernel(x, W, b, dists, neighbors):
    B, T, N, D = x.shape
    H = N_HEADS
    NH = N * H
    xr = x.reshape(B * T, N, D)
    b2 = b.reshape(1, D)
    n_chunks = (B * T) // TCHUNK

    body = functools.partial(_gcn_kernel, T)
    out = pl.pallas_call(
        body,
        grid=(n_chunks,),
        in_specs=[
            pl.BlockSpec((TCHUNK, N, D), lambda c: (c, 0, 0)),
            pl.BlockSpec((D, D), lambda c: (0, 0)),
            pl.BlockSpec((1, D), lambda c: (0, 0)),
            pl.BlockSpec(dists.shape, lambda c: (0, 0)),
            pl.BlockSpec(neighbors.shape, lambda c: (0, 0)),
        ],
        out_specs=pl.BlockSpec((TCHUNK, NH, D), lambda c: (c, 0, 0)),
        out_shape=jax.ShapeDtypeStruct((B * T, NH, D), jnp.float32),
        scratch_shapes=[
            pltpu.VMEM((NH, N), jnp.float32),
            pltpu.VMEM((NH, D), jnp.float32),
            pltpu.VMEM((N, TCHUNK * D), jnp.float32),
            pltpu.VMEM((NH, TCHUNK * D), jnp.float32),
        ],
    )(xr, W, b2, dists, neighbors)
    return out.reshape(B, T, N, H, D)


# speedup vs baseline: 9.1947x; 1.2565x over previous
import functools

import jax
import jax.numpy as jnp
from jax import lax
from jax.experimental import pallas as pl
from jax.experimental.pallas import tpu as pltpu

N_HEADS = 4
SIGMA = 6.0
ALPHA = 0.2
TCHUNK = 32


def _gcn_kernel(T, x_ref, w_ref, b_ref, d_ref, nbr_ref, out_ref,
                a_scr, prev_scr, y_scr, agg_scr):
    c = pl.program_id(0)
    N, K = d_ref.shape
    NH = N * N_HEADS
    D = w_ref.shape[0]

    @pl.when(c == 0)
    def _build_a():
        r_row = lax.broadcasted_iota(jnp.int32, (NH, N), 0)
        i_col = lax.broadcasted_iota(jnp.int32, (NH, N), 1)
        rep = ((r_row // N_HEADS) == i_col).astype(jnp.float32)
        d_rep = jnp.dot(rep, d_ref[...], preferred_element_type=jnp.float32)
        nbr_rep = jnp.dot(rep, nbr_ref[...].astype(jnp.float32),
                          preferred_element_type=jnp.float32)
        lam = ((lax.broadcasted_iota(jnp.int32, (NH, 1), 0) % N_HEADS) + 1
               ).astype(jnp.float32) * (1.0 / N_HEADS)
        n_f = lax.broadcasted_iota(jnp.int32, (NH, N), 1).astype(jnp.float32)
        acc = jnp.zeros((NH, N), dtype=jnp.float32)
        inv_s2 = 1.0 / (SIGMA * SIGMA)
        for k in range(K):
            wgt = jnp.exp(-(d_rep[:, k:k + 1] ** 2) * lam * inv_s2)
            acc = acc + wgt * (nbr_rep[:, k:k + 1] == n_f).astype(jnp.float32)
        a_scr[...] = acc

    x_all = x_ref[...].reshape(TCHUNK * N, D)
    y_stack = lax.dot_general(x_all, w_ref[...], (((1,), (1,)), ((), ())),
                              preferred_element_type=jnp.float32)
    for t in range(TCHUNK):
        y_scr[:, t * D:(t + 1) * D] = y_stack[t * N:(t + 1) * N, :]

    agg_scr[...] = jnp.dot(a_scr[...], y_scr[...],
                           preferred_element_type=jnp.float32)

    bias = b_ref[0]
    is_start = (c * TCHUNK) % T == 0
    prev = prev_scr[...]
    for t in range(TCHUNK):
        cur = agg_scr[:, t * D:(t + 1) * D]
        sm = (1.0 - ALPHA) * cur + ALPHA * prev
        if t == 0:
            sm = jnp.where(is_start, cur, sm)
        out_ref[t] = jnp.maximum(sm + bias[None, :], 0.0)
        prev = cur
    prev_scr[...] = prev


def kernel(x, W, b, dists, neighbors):
    B, T, N, D = x.shape
    H = N_HEADS
    NH = N * H
    xr = x.reshape(B * T, N, D)
    b2 = b.reshape(1, D)
    n_chunks = (B * T) // TCHUNK

    body = functools.partial(_gcn_kernel, T)
    out = pl.pallas_call(
        body,
        grid=(n_chunks,),
        in_specs=[
            pl.BlockSpec((TCHUNK, N, D), lambda c: (c, 0, 0)),
            pl.BlockSpec((D, D), lambda c: (0, 0)),
            pl.BlockSpec((1, D), lambda c: (0, 0)),
            pl.BlockSpec(dists.shape, lambda c: (0, 0)),
            pl.BlockSpec(neighbors.shape, lambda c: (0, 0)),
        ],
        out_specs=pl.BlockSpec((TCHUNK, NH, D), lambda c: (c, 0, 0)),
        out_shape=jax.ShapeDtypeStruct((B * T, NH, D), jnp.float32),
        scratch_shapes=[
            pltpu.VMEM((NH, N), jnp.float32),
            pltpu.VMEM((NH, D), jnp.float32),
            pltpu.VMEM((N, TCHUNK * D), jnp.float32),
            pltpu.VMEM((NH, TCHUNK * D), jnp.float32),
        ],
    )(xr, W, b2, dists, neighbors)
    return out.reshape(B, T, N, H, D)


# speedup vs baseline: 9.4448x; 1.0272x over previous
import functools

import jax
import jax.numpy as jnp
from jax import lax
from jax.experimental import pallas as pl
from jax.experimental.pallas import tpu as pltpu

N_HEADS = 4
SIGMA = 6.0
ALPHA = 0.2


def _gcn_kernel(T, x_ref, w_ref, b_ref, d_ref, nbr_ref, out_ref,
                a_scr, y_scr, agg_scr):
    c = pl.program_id(0)
    N, K = d_ref.shape
    NH = N * N_HEADS
    D = w_ref.shape[0]

    @pl.when(c == 0)
    def _build_a():
        r_row = lax.broadcasted_iota(jnp.int32, (NH, N), 0)
        i_col = lax.broadcasted_iota(jnp.int32, (NH, N), 1)
        rep = ((r_row // N_HEADS) == i_col).astype(jnp.float32)
        d_rep = jnp.dot(rep, d_ref[...], preferred_element_type=jnp.float32)
        nbr_rep = jnp.dot(rep, nbr_ref[...].astype(jnp.float32),
                          preferred_element_type=jnp.float32)
        lam = ((lax.broadcasted_iota(jnp.int32, (NH, 1), 0) % N_HEADS) + 1
               ).astype(jnp.float32) * (1.0 / N_HEADS)
        n_f = lax.broadcasted_iota(jnp.int32, (NH, N), 1).astype(jnp.float32)
        acc = jnp.zeros((NH, N), dtype=jnp.float32)
        inv_s2 = 1.0 / (SIGMA * SIGMA)
        for k in range(K):
            wgt = jnp.exp(-(d_rep[:, k:k + 1] ** 2) * lam * inv_s2)
            acc = acc + wgt * (nbr_rep[:, k:k + 1] == n_f).astype(jnp.float32)
        a_scr[...] = acc

    x_all = x_ref[0].reshape(T * N, D)
    y_stack = lax.dot_general(x_all, w_ref[...], (((1,), (1,)), ((), ())),
                              preferred_element_type=jnp.float32)
    prev = None
    for t in range(T):
        cur = y_stack[t * N:(t + 1) * N, :]
        if t == 0:
            y_scr[:, :D] = cur
        else:
            y_scr[:, t * D:(t + 1) * D] = (1.0 - ALPHA) * cur + ALPHA * prev
        prev = cur

    agg_scr[...] = jnp.dot(a_scr[...], y_scr[...],
                           preferred_element_type=jnp.float32)

    bias = b_ref[0]
    for t in range(T):
        out_ref[0, t] = jnp.maximum(agg_scr[:, t * D:(t + 1) * D] + bias[None, :],
                                    0.0)


def kernel(x, W, b, dists, neighbors):
    B, T, N, D = x.shape
    H = N_HEADS
    NH = N * H
    b2 = b.reshape(1, D)

    body = functools.partial(_gcn_kernel, T)
    out = pl.pallas_call(
        body,
        grid=(B,),
        in_specs=[
            pl.BlockSpec((1, T, N, D), lambda c: (c, 0, 0, 0)),
            pl.BlockSpec((D, D), lambda c: (0, 0)),
            pl.BlockSpec((1, D), lambda c: (0, 0)),
            pl.BlockSpec(dists.shape, lambda c: (0, 0)),
            pl.BlockSpec(neighbors.shape, lambda c: (0, 0)),
        ],
        out_specs=pl.BlockSpec((1, T, NH, D), lambda c: (c, 0, 0, 0)),
        out_shape=jax.ShapeDtypeStruct((B, T, NH, D), jnp.float32),
        scratch_shapes=[
            pltpu.VMEM((NH, N), jnp.float32),
            pltpu.VMEM((N, T * D), jnp.float32),
            pltpu.VMEM((NH, T * D), jnp.float32),
        ],
    )(x, W, b2, dists, neighbors)
    return out.reshape(B, T, N, H, D)
